# baseline (speedup 1.0000x reference)
import functools

import jax
import jax.numpy as jnp
from jax import lax
from jax.experimental import pallas as pl
from jax.experimental.pallas import tpu as pltpu
from jax.experimental.pallas import tpu_sc as plsc

N_FIELDS = 26
CARD = 100000
DIM = 24
BATCH = 16384
OUT_W = N_FIELDS * DIM

NC = 2
NS = 16
NW = NC * NS
ROWS_B = BATCH // NW
IDX_W = ROWS_B * N_FIELDS
BLK = 16
NBLK = ROWS_B // BLK
BLK_IDX = BLK * N_FIELDS

_mesh = plsc.VectorSubcoreMesh(core_axis_name="c", subcore_axis_name="s")


@functools.partial(
    pl.kernel,
    mesh=_mesh,
    out_type=jax.ShapeDtypeStruct((BATCH * N_FIELDS, DIM), jnp.float32),
    scratch_types=[
        pltpu.VMEM((N_FIELDS, ROWS_B), jnp.int32),
        pltpu.VMEM((2, BLK_IDX, DIM), jnp.float32),
        pltpu.SemaphoreType.DMA,
        pltpu.SemaphoreType.DMA,
    ],
)
def _embed_gather(xt_ref, table_ref, out_ref, idx_v, buf_v, gsem, osem):
    wid = lax.axis_index("s") * NC + lax.axis_index("c")
    bbase = pl.multiple_of(wid * ROWS_B, ROWS_B)

    pltpu.sync_copy(xt_ref.at[:, pl.ds(bbase, ROWS_B)], idx_v)

    def do_block(blk, _):
        buf = buf_v.at[lax.rem(blk, 2)]
        @pl.when(blk >= 2)
        def _wait_store():
            pltpu.make_async_copy(
                out_ref.at[pl.ds(0, BLK_IDX), :], buf, osem
            ).wait()

        r0 = pl.multiple_of(blk * BLK, BLK)
        for i in range(N_FIELDS):
            vec = idx_v[i, pl.ds(r0, BLK)]
            for l in range(BLK):
                row = vec[l]
                pltpu.async_copy(
                    table_ref.at[i, pl.ds(row, 1), :],
                    buf.at[pl.ds(l * N_FIELDS + i, 1), :],
                    gsem,
                )
        pltpu.make_async_copy(out_ref.at[pl.ds(0, BLK_IDX), :], buf, gsem).wait()
        o = pl.multiple_of((bbase + r0) * N_FIELDS, 8)
        pltpu.make_async_copy(buf, out_ref.at[pl.ds(o, BLK_IDX), :], osem).start()
        return _

    lax.fori_loop(0, NBLK, do_block, None)
    pltpu.make_async_copy(
        out_ref.at[pl.ds(0, BLK_IDX), :], buf_v.at[0], osem
    ).wait()
    pltpu.make_async_copy(
        out_ref.at[pl.ds(0, BLK_IDX), :], buf_v.at[1], osem
    ).wait()


def kernel(x_cat, tables):
    xt = x_cat.T
    out = _embed_gather(xt, tables)
    return out.reshape(BATCH, OUT_W)
